# baseline (speedup 1.0000x reference)
import functools

import jax
import jax.numpy as jnp
from jax import lax
from jax.experimental import pallas as pl
from jax.experimental.pallas import tpu as pltpu
from jax.experimental.pallas import tpu_sc as plsc


def _bond_mlp_all(edge_attr, W1, b1, W2, b2, *, block_e=4000):
  E, DE = edge_attr.shape
  L, _, D = W1.shape
  GE = E // block_e

  def body(ea_ref, w1_ref, b1_ref, w2_ref, b2_ref, o_ref):
    ea = ea_ref[...].astype(jnp.bfloat16)
    mid = jnp.dot(ea, w1_ref[0], preferred_element_type=jnp.float32)
    mid = jnp.maximum(mid + b1_ref[0], 0.0).astype(jnp.bfloat16)
    out = jnp.dot(mid, w2_ref[0], preferred_element_type=jnp.float32)
    o_ref[0] = out + b2_ref[0]

  return pl.pallas_call(
      body,
      grid=(L, GE),
      in_specs=[
          pl.BlockSpec((block_e, DE), lambda l, i: (i, 0)),
          pl.BlockSpec((1, DE, D), lambda l, i: (l, 0, 0)),
          pl.BlockSpec((1, 1, D), lambda l, i: (l, 0, 0)),
          pl.BlockSpec((1, D, D), lambda l, i: (l, 0, 0)),
          pl.BlockSpec((1, 1, D), lambda l, i: (l, 0, 0)),
      ],
      out_specs=pl.BlockSpec((1, block_e, D), lambda l, i: (l, i, 0)),
      out_shape=jax.ShapeDtypeStruct((L, E, D), jnp.float32),
  )(edge_attr, W1.astype(jnp.bfloat16), b1.reshape(L, 1, D),
    W2.astype(jnp.bfloat16), b2.reshape(L, 1, D))


def _make_sc_aggregate(N, E, D, CH=64):
  info = plsc.get_sparse_core_info()
  NC, NS, LN = info.num_cores, info.num_subcores, info.num_lanes
  NW = NC * NS
  assert E % CH == 0
  NCHUNKS = E // CH
  NFULL = N // CH
  NREM = N - NFULL * CH
  NRT = -(-(NFULL + (1 if NREM else 0)) // NS)
  mesh = plsc.VectorSubcoreMesh(core_axis_name="c", subcore_axis_name="s")

  KMAX = -(-NCHUNKS // NW)
  KMAX += (-KMAX) % 4

  @functools.partial(
      pl.kernel,
      out_type=jax.ShapeDtypeStruct((2, N, D), jnp.float32),
      mesh=mesh,
      scratch_types=[
          pltpu.VMEM((4, 2, CH), jnp.int32),
          pltpu.VMEM((2, CH), jnp.float32),
          pltpu.VMEM((4, CH, D), jnp.float32),
          pltpu.VMEM((2, CH, D), jnp.float32),
          pltpu.VMEM((4, CH), jnp.int32),
          pltpu.VMEM_SHARED((N, D), jnp.float32),
          [pltpu.SemaphoreType.DMA] * 4,
          [pltpu.SemaphoreType.DMA] * 2,
          [pltpu.SemaphoreType.DMA] * 4,
          [pltpu.SemaphoreType.DMA] * 4,
      ],
  )
  def sc_kernel(h_hbm, e_hbm, ew_hbm, pk_hbm, out_hbm,
                pk_v, ew_v, h_v, e_v, dstc_v, acc,
                semP, semE, semG, semS):
    c = lax.axis_index("c")
    s = lax.axis_index("s")
    wid = s * NC + c

    def zrow(i, carry):
      for j in range(D // LN):
        h_v[0, i, pl.ds(j * LN, LN)] = jnp.zeros((LN,), jnp.float32)
      return carry
    lax.fori_loop(0, CH, zrow, 0)
    for t in range(NRT):
      rc = s + NS * t

      @pl.when(rc < NFULL)
      def _():
        pltpu.sync_copy(h_v.at[0], acc.at[pl.ds(rc * CH, CH)])
      if NREM:
        @pl.when(rc == NFULL)
        def _():
          pltpu.sync_copy(h_v.at[0, pl.ds(0, NREM)],
                          acc.at[pl.ds(NFULL * CH, NREM)])
    plsc.subcore_barrier()

    ntw = (NCHUNKS - wid + NW - 1) // NW

    def issue_pk(k, b):
      pltpu.async_copy(pk_hbm.at[wid + k * NW], pk_v.at[b], semP[b])

    def drain_pk(b):
      pltpu.make_async_copy(pk_hbm.at[0], pk_v.at[b], semP[b]).wait()

    def issue_e(k, b):
      base = (wid + k * NW) * CH
      pltpu.async_copy(e_hbm.at[pl.ds(base, CH)], e_v.at[b], semE[b])
      pltpu.async_copy(ew_hbm.at[pl.ds(base, CH)], ew_v.at[b], semE[b])

    def drain_e(b):
      pltpu.make_async_copy(e_hbm.at[pl.ds(0, CH)], e_v.at[b],
                            semE[b]).wait()
      pltpu.make_async_copy(ew_hbm.at[pl.ds(0, CH)], ew_v.at[b],
                            semE[b]).wait()

    def issue_gather(b):
      pltpu.async_copy(h_hbm.at[pk_v.at[b, 0]], h_v.at[b], semG[b])

    def drain_gather(b):
      pltpu.make_async_copy(h_hbm.at[pk_v.at[b, 0]], h_v.at[b],
                            semG[b]).wait()

    def issue_scatter(b):
      pltpu.async_copy(h_v.at[b], acc.at[dstc_v.at[b]], semS[b], add=True)

    def drain_scatter(b):
      pltpu.make_async_copy(h_v.at[b], acc.at[dstc_v.at[b]],
                            semS[b]).wait()

    def compute(b4, b2):
      def edge_body(g, icarry):
        ewv = ew_v[b2, pl.ds(g * LN, LN)]
        dstc_v[b4, pl.ds(g * LN, LN)] = pk_v[b4, 1, pl.ds(g * LN, LN)]
        for ee in range(LN):
          w = ewv[ee]
          row = g * LN + ee

          def col_body(j, jcarry):
            hv = h_v[b4, row, pl.ds(j * LN, LN)]
            ev = e_v[b2, row, pl.ds(j * LN, LN)]
            h_v[b4, row, pl.ds(j * LN, LN)] = jnp.maximum(hv + ev, 0.0) * w
            return jcarry
          lax.fori_loop(0, D // LN, col_body, 0, unroll=4)
        return icarry
      lax.fori_loop(0, CH // LN, edge_body, 0)

    for m in range(3):
      @pl.when(m < ntw)
      def _():
        issue_pk(m, m)
    @pl.when(0 < ntw)
    def _():
      issue_e(0, 0)
      drain_pk(0)
      issue_gather(0)

    def quad_body(k0, carry):
      for b in range(4):
        k = 4 * k0 + b
        b2 = b % 2

        @pl.when(k + 3 < ntw)
        def _():
          issue_pk(k + 3, (b + 3) % 4)

        @pl.when(k + 1 < ntw)
        def _():
          issue_e(k + 1, 1 - b2)

        @pl.when((k >= 3) & (k + 1 < ntw))
        def _():
          drain_scatter((b + 1) % 4)

        @pl.when(k + 1 < ntw)
        def _():
          drain_pk((b + 1) % 4)
          issue_gather((b + 1) % 4)

        @pl.when(k < ntw)
        def _():
          drain_gather(b)
          drain_e(b2)
          compute(b, b2)
          issue_scatter(b)
      return carry
    lax.fori_loop(0, KMAX // 4, quad_body, 0)
    for b in range(4):
      drain_scatter(b)
    plsc.subcore_barrier()

    for t in range(NRT):
      rc = s + NS * t

      @pl.when(rc < NFULL)
      def _():
        pltpu.sync_copy(acc.at[pl.ds(rc * CH, CH)],
                        out_hbm.at[c, pl.ds(rc * CH, CH)])
      if NREM:
        @pl.when(rc == NFULL)
        def _():
          pltpu.sync_copy(acc.at[pl.ds(NFULL * CH, NREM)],
                          out_hbm.at[c, pl.ds(NFULL * CH, NREM)])

  return sc_kernel


def _node_update(h, parts, eps_i, W1, b1, W2, b2, gamma, beta, *, first):
  N, D = h.shape

  def body(h_ref, p_ref, eps_ref, w1_ref, b1_ref, w2_ref, b2_ref,
           g_ref, be_ref, o_ref):
    z = (1.0 + eps_ref[0, 0]) * h_ref[...] + p_ref[0] + p_ref[1]
    z = jnp.dot(z, w1_ref[...], preferred_element_type=jnp.float32)
    z = jnp.maximum(z + b1_ref[...], 0.0)
    z = jnp.dot(z, w2_ref[...], preferred_element_type=jnp.float32)
    z = z + b2_ref[...]
    mean = jnp.mean(z, axis=0, keepdims=True)
    var = jnp.mean(jnp.square(z - mean), axis=0, keepdims=True)
    z = (z - mean) / jnp.sqrt(var + 1e-5) * g_ref[...] + be_ref[...]
    z = jnp.maximum(z, 0.0)
    if first:
      o_ref[...] = z
    else:
      o_ref[...] = h_ref[...] + z

  return pl.pallas_call(
      body,
      out_shape=jax.ShapeDtypeStruct((N, D), jnp.float32),
  )(h, parts, eps_i.reshape(1, 1), W1, b1.reshape(1, D), W2, b2.reshape(1, D),
    gamma.reshape(1, D), beta.reshape(1, D))


def kernel(x, edge_index, edge_attr, edge_weight, eps,
           bond_W1, bond_b1, bond_W2, bond_b2,
           nn_W1, nn_b1, nn_W2, nn_b2, bn_gamma, bn_beta):
  N, D = x.shape
  E = edge_index.shape[1]
  L = bond_W1.shape[0]
  sc_aggregate = _make_sc_aggregate(N, E, D)

  CH = 64
  pk = jnp.stack([
      edge_index[0].reshape(E // CH, CH),
      edge_index[1].reshape(E // CH, CH),
  ], axis=1)

  e_all = _bond_mlp_all(edge_attr, bond_W1, bond_b1, bond_W2, bond_b2)
  h = x
  for i in range(L):
    parts = sc_aggregate(h, e_all[i], edge_weight, pk)
    h = _node_update(h, parts, eps[i], nn_W1[i], nn_b1[i], nn_W2[i], nn_b2[i],
                     bn_gamma[i], bn_beta[i], first=(i == 0))
  return h


# speedup vs baseline: 1.4513x; 1.4513x over previous
import functools

import jax
import jax.numpy as jnp
from jax import lax
from jax.experimental import pallas as pl
from jax.experimental.pallas import tpu as pltpu
from jax.experimental.pallas import tpu_sc as plsc


def _bond_mlp(edge_attr, W1, b1, W2, b2, *, block_e=4000):
  E, DE = edge_attr.shape
  D = W1.shape[1]
  grid = E // block_e

  def body(ea_ref, w1_ref, b1_ref, w2_ref, b2_ref, o_ref):
    ea = ea_ref[...].astype(jnp.bfloat16)
    mid = jnp.dot(ea, w1_ref[...], preferred_element_type=jnp.float32)
    mid = jnp.maximum(mid + b1_ref[...], 0.0).astype(jnp.bfloat16)
    out = jnp.dot(mid, w2_ref[...], preferred_element_type=jnp.float32)
    o_ref[...] = out + b2_ref[...]

  return pl.pallas_call(
      body,
      grid=(grid,),
      in_specs=[
          pl.BlockSpec((block_e, DE), lambda i: (i, 0)),
          pl.BlockSpec((DE, D), lambda i: (0, 0)),
          pl.BlockSpec((1, D), lambda i: (0, 0)),
          pl.BlockSpec((D, D), lambda i: (0, 0)),
          pl.BlockSpec((1, D), lambda i: (0, 0)),
      ],
      out_specs=pl.BlockSpec((block_e, D), lambda i: (i, 0)),
      out_shape=jax.ShapeDtypeStruct((E, D), jnp.float32),
  )(edge_attr, W1.astype(jnp.bfloat16), b1.reshape(1, D),
    W2.astype(jnp.bfloat16), b2.reshape(1, D))


def _make_sc_aggregate(N, E, D, CH=64):
  info = plsc.get_sparse_core_info()
  NC, NS, LN = info.num_cores, info.num_subcores, info.num_lanes
  NW = NC * NS
  assert E % CH == 0
  NCHUNKS = E // CH
  NFULL = N // CH
  NREM = N - NFULL * CH
  NRT = -(-(NFULL + (1 if NREM else 0)) // NS)
  mesh = plsc.VectorSubcoreMesh(core_axis_name="c", subcore_axis_name="s")

  KMAX = -(-NCHUNKS // NW)
  KMAX += (-KMAX) % 4

  @functools.partial(
      pl.kernel,
      out_type=jax.ShapeDtypeStruct((2, N, D), jnp.float32),
      mesh=mesh,
      scratch_types=[
          pltpu.VMEM((4, 2, CH), jnp.int32),
          pltpu.VMEM((2, CH), jnp.float32),
          pltpu.VMEM((4, CH, D), jnp.float32),
          pltpu.VMEM((2, CH, D), jnp.float32),
          pltpu.VMEM((4, CH), jnp.int32),
          pltpu.VMEM_SHARED((N, D), jnp.float32),
          [pltpu.SemaphoreType.DMA] * 4,
          [pltpu.SemaphoreType.DMA] * 2,
          [pltpu.SemaphoreType.DMA] * 4,
          [pltpu.SemaphoreType.DMA] * 4,
      ],
  )
  def sc_kernel(h_hbm, e_hbm, ew_hbm, pk_hbm, out_hbm,
                pk_v, ew_v, h_v, e_v, dstc_v, acc,
                semP, semE, semG, semS):
    c = lax.axis_index("c")
    s = lax.axis_index("s")
    wid = s * NC + c

    def zrow(i, carry):
      for j in range(D // LN):
        h_v[0, i, pl.ds(j * LN, LN)] = jnp.zeros((LN,), jnp.float32)
      return carry
    lax.fori_loop(0, CH, zrow, 0)
    for t in range(NRT):
      rc = s + NS * t

      @pl.when(rc < NFULL)
      def _():
        pltpu.sync_copy(h_v.at[0], acc.at[pl.ds(rc * CH, CH)])
      if NREM:
        @pl.when(rc == NFULL)
        def _():
          pltpu.sync_copy(h_v.at[0, pl.ds(0, NREM)],
                          acc.at[pl.ds(NFULL * CH, NREM)])
    plsc.subcore_barrier()

    ntw = (NCHUNKS - wid + NW - 1) // NW

    def issue_pk(k, b):
      pltpu.async_copy(pk_hbm.at[wid + k * NW], pk_v.at[b], semP[b])

    def drain_pk(b):
      pltpu.make_async_copy(pk_hbm.at[0], pk_v.at[b], semP[b]).wait()

    def issue_e(k, b):
      base = (wid + k * NW) * CH
      pltpu.async_copy(e_hbm.at[pl.ds(base, CH)], e_v.at[b], semE[b])
      pltpu.async_copy(ew_hbm.at[pl.ds(base, CH)], ew_v.at[b], semE[b])

    def drain_e(b):
      pltpu.make_async_copy(e_hbm.at[pl.ds(0, CH)], e_v.at[b],
                            semE[b]).wait()
      pltpu.make_async_copy(ew_hbm.at[pl.ds(0, CH)], ew_v.at[b],
                            semE[b]).wait()

    def issue_gather(b):
      pltpu.async_copy(h_hbm.at[pk_v.at[b, 0]], h_v.at[b], semG[b])

    def drain_gather(b):
      pltpu.make_async_copy(h_hbm.at[pk_v.at[b, 0]], h_v.at[b],
                            semG[b]).wait()

    def issue_scatter(b):
      pltpu.async_copy(h_v.at[b], acc.at[dstc_v.at[b]], semS[b], add=True)

    def drain_scatter(b):
      pltpu.make_async_copy(h_v.at[b], acc.at[dstc_v.at[b]],
                            semS[b]).wait()

    def compute(b4, b2):
      def edge_body(g, icarry):
        ewv = ew_v[b2, pl.ds(g * LN, LN)]
        dstc_v[b4, pl.ds(g * LN, LN)] = pk_v[b4, 1, pl.ds(g * LN, LN)]
        for ee in range(LN):
          w = ewv[ee]
          row = g * LN + ee

          def col_body(j, jcarry):
            hv = h_v[b4, row, pl.ds(j * LN, LN)]
            ev = e_v[b2, row, pl.ds(j * LN, LN)]
            h_v[b4, row, pl.ds(j * LN, LN)] = jnp.maximum(hv + ev, 0.0) * w
            return jcarry
          lax.fori_loop(0, D // LN, col_body, 0, unroll=4)
        return icarry
      lax.fori_loop(0, CH // LN, edge_body, 0)

    for m in range(3):
      @pl.when(m < ntw)
      def _():
        issue_pk(m, m)
    @pl.when(0 < ntw)
    def _():
      issue_e(0, 0)
      drain_pk(0)
      issue_gather(0)

    def quad_body(k0, carry):
      for b in range(4):
        k = 4 * k0 + b
        b2 = b % 2

        @pl.when(k + 3 < ntw)
        def _():
          issue_pk(k + 3, (b + 3) % 4)

        @pl.when(k + 1 < ntw)
        def _():
          issue_e(k + 1, 1 - b2)

        @pl.when((k >= 3) & (k + 1 < ntw))
        def _():
          drain_scatter((b + 1) % 4)

        @pl.when(k + 1 < ntw)
        def _():
          drain_pk((b + 1) % 4)
          issue_gather((b + 1) % 4)

        @pl.when(k < ntw)
        def _():
          drain_gather(b)
          drain_e(b2)
          compute(b, b2)
          issue_scatter(b)
      return carry
    lax.fori_loop(0, KMAX // 4, quad_body, 0)
    for b in range(4):
      drain_scatter(b)
    plsc.subcore_barrier()

    for t in range(NRT):
      rc = s + NS * t

      @pl.when(rc < NFULL)
      def _():
        pltpu.sync_copy(acc.at[pl.ds(rc * CH, CH)],
                        out_hbm.at[c, pl.ds(rc * CH, CH)])
      if NREM:
        @pl.when(rc == NFULL)
        def _():
          pltpu.sync_copy(acc.at[pl.ds(NFULL * CH, NREM)],
                          out_hbm.at[c, pl.ds(NFULL * CH, NREM)])

  return sc_kernel


def _node_update(h, parts, eps_i, W1, b1, W2, b2, gamma, beta, *, first):
  N, D = h.shape

  def body(h_ref, p_ref, eps_ref, w1_ref, b1_ref, w2_ref, b2_ref,
           g_ref, be_ref, o_ref):
    z = (1.0 + eps_ref[0, 0]) * h_ref[...] + p_ref[0] + p_ref[1]
    z = jnp.dot(z, w1_ref[...], preferred_element_type=jnp.float32)
    z = jnp.maximum(z + b1_ref[...], 0.0)
    z = jnp.dot(z, w2_ref[...], preferred_element_type=jnp.float32)
    z = z + b2_ref[...]
    mean = jnp.mean(z, axis=0, keepdims=True)
    var = jnp.mean(jnp.square(z - mean), axis=0, keepdims=True)
    z = (z - mean) / jnp.sqrt(var + 1e-5) * g_ref[...] + be_ref[...]
    z = jnp.maximum(z, 0.0)
    if first:
      o_ref[...] = z
    else:
      o_ref[...] = h_ref[...] + z

  return pl.pallas_call(
      body,
      out_shape=jax.ShapeDtypeStruct((N, D), jnp.float32),
  )(h, parts, eps_i.reshape(1, 1), W1, b1.reshape(1, D), W2, b2.reshape(1, D),
    gamma.reshape(1, D), beta.reshape(1, D))


def kernel(x, edge_index, edge_attr, edge_weight, eps,
           bond_W1, bond_b1, bond_W2, bond_b2,
           nn_W1, nn_b1, nn_W2, nn_b2, bn_gamma, bn_beta):
  N, D = x.shape
  E = edge_index.shape[1]
  L = bond_W1.shape[0]
  sc_aggregate = _make_sc_aggregate(N, E, D)

  CH = 64
  pk = jnp.stack([
      edge_index[0].reshape(E // CH, CH),
      edge_index[1].reshape(E // CH, CH),
  ], axis=1)

  es = [_bond_mlp(edge_attr, bond_W1[i], bond_b1[i], bond_W2[i], bond_b2[i])
        for i in range(L)]
  h = x
  for i in range(L):
    parts = sc_aggregate(h, es[i], edge_weight, pk)
    h = _node_update(h, parts, eps[i], nn_W1[i], nn_b1[i], nn_W2[i], nn_b2[i],
                     bn_gamma[i], bn_beta[i], first=(i == 0))
  return h


# speedup vs baseline: 1.4638x; 1.0086x over previous
import functools

import jax
import jax.numpy as jnp
from jax import lax
from jax.experimental import pallas as pl
from jax.experimental.pallas import tpu as pltpu
from jax.experimental.pallas import tpu_sc as plsc


def _bond_mlp(edge_attr, W1, b1, W2, b2, *, block_e=4000):
  E, DE = edge_attr.shape
  D = W1.shape[1]
  grid = E // block_e

  def body(ea_ref, w1_ref, b1_ref, w2_ref, b2_ref, o_ref):
    ea = ea_ref[...].astype(jnp.bfloat16)
    mid = jnp.dot(ea, w1_ref[...], preferred_element_type=jnp.float32)
    mid = jnp.maximum(mid + b1_ref[...], 0.0).astype(jnp.bfloat16)
    out = jnp.dot(mid, w2_ref[...], preferred_element_type=jnp.float32)
    o_ref[...] = out + b2_ref[...]

  return pl.pallas_call(
      body,
      grid=(grid,),
      in_specs=[
          pl.BlockSpec((block_e, DE), lambda i: (i, 0)),
          pl.BlockSpec((DE, D), lambda i: (0, 0)),
          pl.BlockSpec((1, D), lambda i: (0, 0)),
          pl.BlockSpec((D, D), lambda i: (0, 0)),
          pl.BlockSpec((1, D), lambda i: (0, 0)),
      ],
      out_specs=pl.BlockSpec((block_e, D), lambda i: (i, 0)),
      out_shape=jax.ShapeDtypeStruct((E, D), jnp.float32),
  )(edge_attr, W1.astype(jnp.bfloat16), b1.reshape(1, D),
    W2.astype(jnp.bfloat16), b2.reshape(1, D))


def _make_sc_aggregate(N, E, D, CH=64):
  info = plsc.get_sparse_core_info()
  NC, NS, LN = info.num_cores, info.num_subcores, info.num_lanes
  NW = NC * NS
  assert E % CH == 0
  NCHUNKS = E // CH
  NFULL = N // CH
  NREM = N - NFULL * CH
  NRT = -(-(NFULL + (1 if NREM else 0)) // NS)
  mesh = plsc.VectorSubcoreMesh(core_axis_name="c", subcore_axis_name="s")

  KMAX = -(-NCHUNKS // NW)
  KMAX += (-KMAX) % 4

  @functools.partial(
      pl.kernel,
      out_type=jax.ShapeDtypeStruct((2, N, D), jnp.float32),
      mesh=mesh,
      scratch_types=[
          pltpu.VMEM((4, 2, CH), jnp.int32),
          pltpu.VMEM((2, CH), jnp.float32),
          pltpu.VMEM((4, CH, D), jnp.float32),
          pltpu.VMEM((2, CH, D), jnp.float32),
          pltpu.VMEM((4, CH), jnp.int32),
          pltpu.VMEM_SHARED((N, D), jnp.float32),
          [pltpu.SemaphoreType.DMA] * 4,
          [pltpu.SemaphoreType.DMA] * 2,
          [pltpu.SemaphoreType.DMA] * 4,
          [pltpu.SemaphoreType.DMA] * 4,
      ],
  )
  def sc_kernel(h_hbm, e_hbm, ew_hbm, pk_hbm, out_hbm,
                pk_v, ew_v, h_v, e_v, dstc_v, acc,
                semP, semE, semG, semS):
    c = lax.axis_index("c")
    s = lax.axis_index("s")
    wid = s * NC + c

    def zrow(i, carry):
      for j in range(D // LN):
        h_v[0, i, pl.ds(j * LN, LN)] = jnp.zeros((LN,), jnp.float32)
      return carry
    lax.fori_loop(0, CH, zrow, 0)
    for t in range(NRT):
      rc = s + NS * t

      @pl.when(rc < NFULL)
      def _():
        pltpu.sync_copy(h_v.at[0], acc.at[pl.ds(rc * CH, CH)])
      if NREM:
        @pl.when(rc == NFULL)
        def _():
          pltpu.sync_copy(h_v.at[0, pl.ds(0, NREM)],
                          acc.at[pl.ds(NFULL * CH, NREM)])
    plsc.subcore_barrier()

    ntw = (NCHUNKS - wid + NW - 1) // NW

    def issue_pk(k, b):
      pltpu.async_copy(pk_hbm.at[wid + k * NW], pk_v.at[b], semP[b])

    def drain_pk(b):
      pltpu.make_async_copy(pk_hbm.at[0], pk_v.at[b], semP[b]).wait()

    def issue_e(k, b):
      base = (wid + k * NW) * CH
      pltpu.async_copy(e_hbm.at[pl.ds(base, CH)], e_v.at[b], semE[b])
      pltpu.async_copy(ew_hbm.at[pl.ds(base, CH)], ew_v.at[b], semE[b])

    def drain_e(b):
      pltpu.make_async_copy(e_hbm.at[pl.ds(0, CH)], e_v.at[b],
                            semE[b]).wait()
      pltpu.make_async_copy(ew_hbm.at[pl.ds(0, CH)], ew_v.at[b],
                            semE[b]).wait()

    def issue_gather(b):
      pltpu.async_copy(h_hbm.at[pk_v.at[b, 0]], h_v.at[b], semG[b])

    def drain_gather(b):
      pltpu.make_async_copy(h_hbm.at[pk_v.at[b, 0]], h_v.at[b],
                            semG[b]).wait()

    def issue_scatter(b):
      pltpu.async_copy(h_v.at[b], acc.at[dstc_v.at[b]], semS[b], add=True)

    def drain_scatter(b):
      pltpu.make_async_copy(h_v.at[b], acc.at[dstc_v.at[b]],
                            semS[b]).wait()

    def compute(b4, b2):
      def edge_body(g, icarry):
        ewv = ew_v[b2, pl.ds(g * LN, LN)]
        dstc_v[b4, pl.ds(g * LN, LN)] = pk_v[b4, 1, pl.ds(g * LN, LN)]
        for ee in range(LN):
          w = ewv[ee]
          row = g * LN + ee

          def col_body(j, jcarry):
            hv = h_v[b4, row, pl.ds(j * LN, LN)]
            ev = e_v[b2, row, pl.ds(j * LN, LN)]
            h_v[b4, row, pl.ds(j * LN, LN)] = jnp.maximum(hv + ev, 0.0) * w
            return jcarry
          lax.fori_loop(0, D // LN, col_body, 0, unroll=4)
        return icarry
      lax.fori_loop(0, CH // LN, edge_body, 0)

    for m in range(3):
      @pl.when(m < ntw)
      def _():
        issue_pk(m, m)
    @pl.when(0 < ntw)
    def _():
      issue_e(0, 0)
      drain_pk(0)
      issue_gather(0)
    @pl.when(1 < ntw)
    def _():
      drain_pk(1)
      issue_gather(1)

    def quad_body(k0, carry):
      for b in range(4):
        k = 4 * k0 + b
        b2 = b % 2

        @pl.when(k + 3 < ntw)
        def _():
          issue_pk(k + 3, (b + 3) % 4)

        @pl.when(k + 1 < ntw)
        def _():
          issue_e(k + 1, 1 - b2)

        @pl.when((k >= 2) & (k + 2 < ntw))
        def _():
          drain_scatter((b + 2) % 4)

        @pl.when(k + 2 < ntw)
        def _():
          drain_pk((b + 2) % 4)
          issue_gather((b + 2) % 4)

        @pl.when(k < ntw)
        def _():
          drain_gather(b)
          drain_e(b2)
          compute(b, b2)
          issue_scatter(b)
      return carry
    lax.fori_loop(0, KMAX // 4, quad_body, 0)
    for b in range(4):
      drain_scatter(b)
    plsc.subcore_barrier()

    for t in range(NRT):
      rc = s + NS * t

      @pl.when(rc < NFULL)
      def _():
        pltpu.sync_copy(acc.at[pl.ds(rc * CH, CH)],
                        out_hbm.at[c, pl.ds(rc * CH, CH)])
      if NREM:
        @pl.when(rc == NFULL)
        def _():
          pltpu.sync_copy(acc.at[pl.ds(NFULL * CH, NREM)],
                          out_hbm.at[c, pl.ds(NFULL * CH, NREM)])

  return sc_kernel


def _node_update(h, parts, eps_i, W1, b1, W2, b2, gamma, beta, *, first):
  N, D = h.shape

  def body(h_ref, p_ref, eps_ref, w1_ref, b1_ref, w2_ref, b2_ref,
           g_ref, be_ref, o_ref):
    z = (1.0 + eps_ref[0, 0]) * h_ref[...] + p_ref[0] + p_ref[1]
    z = jnp.dot(z, w1_ref[...], preferred_element_type=jnp.float32)
    z = jnp.maximum(z + b1_ref[...], 0.0)
    z = jnp.dot(z, w2_ref[...], preferred_element_type=jnp.float32)
    z = z + b2_ref[...]
    mean = jnp.mean(z, axis=0, keepdims=True)
    var = jnp.mean(jnp.square(z - mean), axis=0, keepdims=True)
    z = (z - mean) / jnp.sqrt(var + 1e-5) * g_ref[...] + be_ref[...]
    z = jnp.maximum(z, 0.0)
    if first:
      o_ref[...] = z
    else:
      o_ref[...] = h_ref[...] + z

  return pl.pallas_call(
      body,
      out_shape=jax.ShapeDtypeStruct((N, D), jnp.float32),
  )(h, parts, eps_i.reshape(1, 1), W1, b1.reshape(1, D), W2, b2.reshape(1, D),
    gamma.reshape(1, D), beta.reshape(1, D))


def kernel(x, edge_index, edge_attr, edge_weight, eps,
           bond_W1, bond_b1, bond_W2, bond_b2,
           nn_W1, nn_b1, nn_W2, nn_b2, bn_gamma, bn_beta):
  N, D = x.shape
  E = edge_index.shape[1]
  L = bond_W1.shape[0]
  sc_aggregate = _make_sc_aggregate(N, E, D)

  CH = 64
  pk = jnp.stack([
      edge_index[0].reshape(E // CH, CH),
      edge_index[1].reshape(E // CH, CH),
  ], axis=1)

  es = [_bond_mlp(edge_attr, bond_W1[i], bond_b1[i], bond_W2[i], bond_b2[i])
        for i in range(L)]
  h = x
  for i in range(L):
    parts = sc_aggregate(h, es[i], edge_weight, pk)
    h = _node_update(h, parts, eps[i], nn_W1[i], nn_b1[i], nn_W2[i], nn_b2[i],
                     bn_gamma[i], bn_beta[i], first=(i == 0))
  return h
